# baseline (speedup 1.0000x reference)
import functools

import jax
import jax.numpy as jnp
from jax import lax
from jax.experimental import pallas as pl
from jax.experimental.pallas import tpu as pltpu
from jax.experimental.pallas import tpu_sc as plsc

ENT = 10000
DIM = 128
NC = 2
NS = 16
NW = NC * NS
L = 16

H = 5120
AGG_ROWS = 2 * H
AGH = H + 8
TSTRIPE = H // NS

SLICE = 10112
PCH = 128
CCHUNK = 112
CAPQ = 163840
TRASH = H

BATCH = 4096
B_PER_W = BATCH // NW


def _sc_pos_scatter(packed, pos):
    nch = SLICE // PCH
    assert nch % 3 == 1
    mesh = plsc.VectorSubcoreMesh(core_axis_name="c", subcore_axis_name="s")

    @functools.partial(
        pl.kernel,
        out_type=jax.ShapeDtypeStruct((NC * 4 * CAPQ,), jnp.int32),
        mesh=mesh,
        scratch_types=[
            pltpu.VMEM((3, PCH), jnp.int32),
            pltpu.VMEM((3, PCH), jnp.int32),
            pltpu.VMEM((4096,), jnp.int32),
            [pltpu.SemaphoreType.DMA] * 3,
            [pltpu.SemaphoreType.DMA] * 3,
        ],
    )
    def k(pk_hbm, ps_hbm, out_hbm, pk_v, ps_v, tr_v, isems, ssems):
        cid = lax.axis_index("c")
        sid = lax.axis_index("s")
        base_e = (cid * NS + sid) * SLICE

        t16 = jnp.full((L,), TRASH, jnp.int32)

        def tfill(i, c):
            tr_v[pl.ds(i * L, L)] = t16
            return c

        lax.fori_loop(0, 4096 // L, tfill, 0)
        stripe0 = cid * (4 * CAPQ) + sid * (4 * CAPQ // NS)
        for u in range(4 * CAPQ // NS // 4096):
            pltpu.sync_copy(tr_v, out_hbm.at[pl.ds(stripe0 + u * 4096, 4096)])

        def fire_idx(j, b):
            eoff = pl.multiple_of(base_e + j * PCH, 8)
            pltpu.async_copy(pk_hbm.at[pl.ds(eoff, PCH)], pk_v.at[b],
                             isems[b])
            pltpu.async_copy(ps_hbm.at[pl.ds(eoff, PCH)], ps_v.at[b],
                             isems[b])

        def wait_idx(j, b):
            eoff = pl.multiple_of(base_e + j * PCH, 8)
            pltpu.make_async_copy(pk_hbm.at[pl.ds(eoff, PCH)], pk_v.at[b],
                                  isems[b]).wait()
            pltpu.make_async_copy(ps_hbm.at[pl.ds(eoff, PCH)], ps_v.at[b],
                                  isems[b]).wait()

        def fire_scatter(b):
            pltpu.async_copy(pk_v.at[b], out_hbm.at[ps_v.at[b]], ssems[b])

        def wait_scatter(b):
            pltpu.make_async_copy(pk_v.at[b], out_hbm.at[ps_v.at[b]],
                                  ssems[b]).wait()

        plsc.subcore_barrier()
        fire_idx(0, 0)
        fire_idx(1, 1)

        def step(j, b, first=False, fire_i=True):
            if not first:
                wait_scatter((b + 2) % 3)
            if fire_i:
                fire_idx(j + 2, (b + 2) % 3)
            wait_idx(j, b)
            fire_scatter(b)

        step(0, 0, first=True)

        def body3(i, carry):
            jb = 1 + 3 * i
            for u in range(3):
                step(jb + u, (1 + u) % 3)
            return carry

        lax.fori_loop(0, (nch - 4) // 3, body3, 0)
        step(nch - 3, 1)
        step(nch - 2, 2, fire_i=False)
        step(nch - 1, 0, fire_i=False)
        wait_scatter(0)
        plsc.subcore_barrier()

    return k(packed, pos)


def _sc_quadrant_aggregate(table, qarr, qcnt, zeros):
    mesh = plsc.VectorSubcoreMesh(core_axis_name="c", subcore_axis_name="s")

    @functools.partial(
        pl.kernel,
        out_type=jax.ShapeDtypeStruct((NC, 2, H, DIM), jnp.float32),
        mesh=mesh,
        scratch_types=[
            pltpu.VMEM((3, CCHUNK), jnp.int32),
            pltpu.VMEM((3, CCHUNK), jnp.int32),
            pltpu.VMEM((3, CCHUNK), jnp.int32),
            pltpu.VMEM((3, CCHUNK, DIM), jnp.float32),
            pltpu.VMEM((L,), jnp.int32),
            pltpu.VMEM_SHARED((H, DIM), jnp.float32),
            pltpu.VMEM_SHARED((AGH, DIM), jnp.float32),
            [pltpu.SemaphoreType.DMA] * 3,
            [pltpu.SemaphoreType.DMA] * 3,
            [pltpu.SemaphoreType.DMA] * 3,
        ],
    )
    def k(table_hbm, qarr_hbm, qcnt_hbm, zeros_hbm, out_hbm,
          pk_v, src_v, dst_v, rows_v, cnt_v, tab_sh, agg_sh,
          isems, gsems, ssems):
        cid = lax.axis_index("c")
        sid = lax.axis_index("s")
        row0 = pl.multiple_of(sid * TSTRIPE, 8)

        pltpu.sync_copy(table_hbm.at[pl.ds(cid * H + row0, TSTRIPE)],
                        tab_sh.at[pl.ds(row0, TSTRIPE)])
        pltpu.sync_copy(qcnt_hbm.at[pl.ds(cid * L, L)], cnt_v)

        def fire_idx(base_e, j, b):
            eoff = pl.multiple_of(base_e + j * CCHUNK, 8)
            pltpu.async_copy(qarr_hbm.at[pl.ds(eoff, CCHUNK)], pk_v.at[b],
                             isems[b])

        def wait_unpack_idx(base_e, j, b):
            eoff = pl.multiple_of(base_e + j * CCHUNK, 8)
            pltpu.make_async_copy(qarr_hbm.at[pl.ds(eoff, CCHUNK)],
                                  pk_v.at[b], isems[b]).wait()
            for u in range(CCHUNK // L):
                p16 = pk_v[b, pl.ds(u * L, L)]
                src_v[b, pl.ds(u * L, L)] = p16 >> 13
                dst_v[b, pl.ds(u * L, L)] = p16 & (2 ** 13 - 1)

        def fire_gather(b):
            pltpu.async_copy(tab_sh.at[src_v.at[b]], rows_v.at[b], gsems[b])

        def wait_gather(b):
            pltpu.make_async_copy(tab_sh.at[src_v.at[b]], rows_v.at[b],
                                  gsems[b]).wait()

        def fire_scatter(b):
            pltpu.async_copy(rows_v.at[b], agg_sh.at[dst_v.at[b]],
                             ssems[b], add=True)

        def wait_scatter(b):
            pltpu.make_async_copy(rows_v.at[b], agg_sh.at[dst_v.at[b]],
                                  ssems[b]).wait()

        def run_region(base_e, n):
            fire_idx(base_e, 0, 0)
            fire_idx(base_e, 1, 1)
            wait_unpack_idx(base_e, 0, 0)
            fire_gather(0)

            def step(j, b, first=False, fire_i=True, fire_g=True):
                if not first:
                    wait_scatter((b + 2) % 3)
                if fire_i:
                    fire_idx(base_e, j + 2, (b + 2) % 3)
                wait_gather(b)
                fire_scatter(b)
                if fire_g:
                    wait_unpack_idx(base_e, j + 1, (b + 1) % 3)
                    fire_gather((b + 1) % 3)

            step(0, 0, first=True)

            def body3(i, carry):
                jb = 1 + 3 * i
                for u in range(3):
                    step(jb + u, (1 + u) % 3)
                return carry

            lax.fori_loop(0, (n - 4) // 3, body3, 0)
            step(n - 3, 1)
            step(n - 2, 2, fire_i=False)
            step(n - 1, 0, fire_i=False, fire_g=False)
            wait_scatter(0)

        for p in range(2):
            pltpu.sync_copy(zeros_hbm.at[pl.ds(row0, TSTRIPE)],
                            agg_sh.at[pl.ds(row0, TSTRIPE)])
            plsc.subcore_barrier()
            q = cid * 2 + p
            for h in range(2):
                c = cnt_v[...][2 * h + p]
                nq = (c + (CCHUNK - 1)) // CCHUNK
                nt = (nq + (NS - 1)) // NS
                n = nt + (1 - nt) % 3
                n = jnp.maximum(n, 4)
                base_e = (h * 4 + q) * CAPQ + sid * (n * CCHUNK)
                run_region(base_e, n)
            plsc.subcore_barrier()
            pltpu.sync_copy(agg_sh.at[pl.ds(row0, TSTRIPE)],
                            out_hbm.at[cid, p, pl.ds(row0, TSTRIPE)])
            plsc.subcore_barrier()

    return k(table, qarr, qcnt, zeros)


def _tc_combine_matmul(partials, wt, b2):
    def body(p_ref, w_ref, b_ref, o_ref):
        x = jnp.concatenate(
            [p_ref[0, 0] + p_ref[1, 0], p_ref[0, 1] + p_ref[1, 1]], axis=0)
        y = jax.lax.dot_general(
            x, w_ref[:], (((1,), (0,)), ((), ())),
            precision=jax.lax.Precision.HIGHEST,
            preferred_element_type=jnp.float32)
        o_ref[:] = jnp.maximum(y + b_ref[:], 0.0)

    return pl.pallas_call(
        body,
        out_shape=jax.ShapeDtypeStruct((AGG_ROWS, DIM), jnp.float32),
    )(partials, wt, b2)


def _sc_batch_gather(ent, rel_emb, norm_emb, head, rel, tail, time):
    mesh = plsc.VectorSubcoreMesh(core_axis_name="c", subcore_axis_name="s")
    out_sds = jax.ShapeDtypeStruct((BATCH, DIM), jnp.float32)

    @functools.partial(
        pl.kernel,
        out_type=(out_sds, out_sds, out_sds, out_sds),
        mesh=mesh,
        scratch_types=[
            pltpu.VMEM((B_PER_W,), jnp.int32),
            pltpu.VMEM((B_PER_W, DIM), jnp.float32),
            pltpu.SemaphoreType.DMA,
        ],
    )
    def k(ent_hbm, rel_hbm, nrm_hbm, hidx_hbm, ridx_hbm, tidx_hbm, midx_hbm,
          h_out, r_out, t_out, n_out, idx_v, rows_v, sem):
        cid = lax.axis_index("c")
        sid = lax.axis_index("s")
        wid = sid * NC + cid
        base = pl.multiple_of(wid * B_PER_W, B_PER_W)

        def one(idx_hbm, table_hbm, out_hbm):
            pltpu.sync_copy(idx_hbm.at[pl.ds(base, B_PER_W)], idx_v)
            pltpu.async_copy(table_hbm.at[idx_v], rows_v, sem).wait()
            pltpu.sync_copy(rows_v, out_hbm.at[pl.ds(base, B_PER_W)])

        one(hidx_hbm, ent_hbm, h_out)
        one(ridx_hbm, rel_hbm, r_out)
        one(tidx_hbm, ent_hbm, t_out)
        one(midx_hbm, nrm_hbm, n_out)

    return k(ent, rel_emb, norm_emb, head, rel, tail, time)


def _tc_score(h, r, t, nv):
    def body(h_ref, r_ref, t_ref, n_ref, o_ref):
        def normalize(x):
            n = jnp.sqrt(jnp.sum(x * x, axis=-1, keepdims=True))
            return x / jnp.maximum(n, 1e-12)

        nvn = normalize(n_ref[:])

        def proj(e):
            return e - jnp.sum(nvn * e, axis=-1, keepdims=True) * nvn

        hh = normalize(proj(h_ref[:]))
        rr = normalize(proj(r_ref[:]))
        tt = normalize(proj(t_ref[:]))
        d = hh + rr - tt
        o_ref[:] = jnp.sqrt(jnp.sum(d * d, axis=-1, keepdims=True))

    return pl.pallas_call(
        body,
        out_shape=jax.ShapeDtypeStruct((BATCH, 1), jnp.float32),
    )(h, r, t, nv)


def kernel(feature, edge_index, head_batched, rel_batched, tail_batched,
           time_batched, W, b, rel_emb, norm_emb):
    n_edges = edge_index.shape[1]
    e_pad = NW * SLICE
    e_half = NS * SLICE
    assert n_edges <= e_pad
    pad = e_pad - n_edges
    src2 = jnp.concatenate([edge_index[0], jnp.zeros((pad,), jnp.int32)])
    dst2 = jnp.concatenate([edge_index[1], jnp.full((pad,), 2 * H, jnp.int32)])

    qa = (src2 >= H).astype(jnp.int32).reshape(NC, e_half)
    qb = (dst2 >= H).astype(jnp.int32).reshape(NC, e_half)
    ind = [(1 - qa) * (1 - qb), (1 - qa) * qb, qa * (1 - qb), qa * qb]
    ranks = [jnp.cumsum(x, axis=1, dtype=jnp.int32) for x in ind]
    qq = qa * 2 + qb
    rsel = jnp.where(qq == 0, ranks[0], jnp.where(qq == 1, ranks[1],
                     jnp.where(qq == 2, ranks[2], ranks[3])))
    plane = jnp.arange(NC, dtype=jnp.int32).reshape(NC, 1)
    pos = (plane * (4 * CAPQ) + qq * CAPQ + rsel - 1).reshape(-1)
    sloc = src2 - qa.reshape(-1) * H
    dloc = dst2 - qb.reshape(-1) * H
    packed = (sloc << 13) | dloc
    qcnt = jnp.concatenate([
        jnp.concatenate(
            [jnp.stack([ranks[2 * c + p_][h, -1]
                        for h in range(2) for p_ in range(2)]),
             jnp.zeros((L - 4,), jnp.int32)])
        for c in range(NC)])

    zeros = jnp.zeros((H, DIM), jnp.float32)
    wt = W.T
    b2 = b.reshape(1, DIM)
    feature_p = jnp.concatenate(
        [feature, jnp.zeros((AGG_ROWS - ENT, DIM), jnp.float32)])

    qarr = _sc_pos_scatter(packed, pos)
    p1 = _sc_quadrant_aggregate(feature_p, qarr, qcnt, zeros)
    f1 = _tc_combine_matmul(p1, wt, b2)
    p2 = _sc_quadrant_aggregate(f1, qarr, qcnt, zeros)
    ent = _tc_combine_matmul(p2, wt, b2)

    h, r, t, nv = _sc_batch_gather(ent, rel_emb, norm_emb, head_batched,
                                   rel_batched, tail_batched, time_batched)
    return _tc_score(h, r, t, nv).reshape(-1)


# speedup vs baseline: 2.2323x; 2.2323x over previous
import functools

import jax
import jax.numpy as jnp
from jax import lax
from jax.experimental import pallas as pl
from jax.experimental.pallas import tpu as pltpu
from jax.experimental.pallas import tpu_sc as plsc

ENT = 10000
DIM = 128
NC = 2
NS = 16
NW = NC * NS
L = 16

H = 5120
AGG_ROWS = 2 * H
AGH = H + 8
TSTRIPE = H // NS

SLICE = 10112
PCH = 128
CCHUNK = 112
CAPQ = 163840
TRASH = H

BATCH = 4096
B_PER_W = BATCH // NW


def _sc_pos_scatter(packed, pos, trash):
    nch = SLICE // PCH
    assert nch % 3 == 1
    mesh = plsc.VectorSubcoreMesh(core_axis_name="c", subcore_axis_name="s")

    @functools.partial(
        pl.kernel,
        out_type=jax.ShapeDtypeStruct((NC * 4 * CAPQ,), jnp.int32),
        mesh=mesh,
        scratch_types=[
            pltpu.VMEM((3, PCH), jnp.int32),
            pltpu.VMEM((3, PCH), jnp.int32),
            pltpu.VMEM_SHARED((4 * CAPQ,), jnp.int32),
            [pltpu.SemaphoreType.DMA] * 3,
            [pltpu.SemaphoreType.DMA] * 3,
        ],
    )
    def k(pk_hbm, ps_hbm, tr_hbm, out_hbm, pk_v, ps_v, plane_sh,
          isems, ssems):
        cid = lax.axis_index("c")
        sid = lax.axis_index("s")
        base_e = (cid * NS + sid) * SLICE
        stripe = 4 * CAPQ // NS
        s0 = sid * stripe

        def fire_idx(j, b):
            eoff = pl.multiple_of(base_e + j * PCH, 8)
            pltpu.async_copy(pk_hbm.at[pl.ds(eoff, PCH)], pk_v.at[b],
                             isems[b])
            pltpu.async_copy(ps_hbm.at[pl.ds(eoff, PCH)], ps_v.at[b],
                             isems[b])

        def wait_idx(j, b):
            eoff = pl.multiple_of(base_e + j * PCH, 8)
            pltpu.make_async_copy(pk_hbm.at[pl.ds(eoff, PCH)], pk_v.at[b],
                                  isems[b]).wait()
            pltpu.make_async_copy(ps_hbm.at[pl.ds(eoff, PCH)], ps_v.at[b],
                                  isems[b]).wait()

        def fire_scatter(b):
            pltpu.async_copy(pk_v.at[b], plane_sh.at[ps_v.at[b]], ssems[b])

        def wait_scatter(b):
            pltpu.make_async_copy(pk_v.at[b], plane_sh.at[ps_v.at[b]],
                                  ssems[b]).wait()

        fire_idx(0, 0)
        fire_idx(1, 1)
        pltpu.sync_copy(tr_hbm, plane_sh.at[pl.ds(s0, stripe)])
        plsc.subcore_barrier()

        def step(j, b, first=False, fire_i=True):
            if not first:
                wait_scatter((b + 2) % 3)
            if fire_i:
                fire_idx(j + 2, (b + 2) % 3)
            wait_idx(j, b)
            fire_scatter(b)

        step(0, 0, first=True)

        def body3(i, carry):
            jb = 1 + 3 * i
            for u in range(3):
                step(jb + u, (1 + u) % 3)
            return carry

        lax.fori_loop(0, (nch - 4) // 3, body3, 0)
        step(nch - 3, 1)
        step(nch - 2, 2, fire_i=False)
        step(nch - 1, 0, fire_i=False)
        wait_scatter(0)
        plsc.subcore_barrier()
        pltpu.sync_copy(plane_sh.at[pl.ds(s0, stripe)],
                        out_hbm.at[pl.ds(cid * (4 * CAPQ) + s0, stripe)])

    return k(packed, pos, trash)


def _sc_quadrant_aggregate(table, qarr, qcnt, zeros):
    mesh = plsc.VectorSubcoreMesh(core_axis_name="c", subcore_axis_name="s")

    @functools.partial(
        pl.kernel,
        out_type=jax.ShapeDtypeStruct((NC, 2, H, DIM), jnp.float32),
        mesh=mesh,
        scratch_types=[
            pltpu.VMEM((3, CCHUNK), jnp.int32),
            pltpu.VMEM((3, CCHUNK), jnp.int32),
            pltpu.VMEM((3, CCHUNK), jnp.int32),
            pltpu.VMEM((3, CCHUNK, DIM), jnp.float32),
            pltpu.VMEM((L,), jnp.int32),
            pltpu.VMEM_SHARED((H, DIM), jnp.float32),
            pltpu.VMEM_SHARED((AGH, DIM), jnp.float32),
            [pltpu.SemaphoreType.DMA] * 3,
            [pltpu.SemaphoreType.DMA] * 3,
            [pltpu.SemaphoreType.DMA] * 3,
        ],
    )
    def k(table_hbm, qarr_hbm, qcnt_hbm, zeros_hbm, out_hbm,
          pk_v, src_v, dst_v, rows_v, cnt_v, tab_sh, agg_sh,
          isems, gsems, ssems):
        cid = lax.axis_index("c")
        sid = lax.axis_index("s")
        row0 = pl.multiple_of(sid * TSTRIPE, 8)

        pltpu.sync_copy(table_hbm.at[pl.ds(cid * H + row0, TSTRIPE)],
                        tab_sh.at[pl.ds(row0, TSTRIPE)])
        pltpu.sync_copy(qcnt_hbm.at[pl.ds(cid * L, L)], cnt_v)

        def fire_idx(base_e, j, b):
            eoff = pl.multiple_of(base_e + j * CCHUNK, 8)
            pltpu.async_copy(qarr_hbm.at[pl.ds(eoff, CCHUNK)], pk_v.at[b],
                             isems[b])

        def wait_unpack_idx(base_e, j, b):
            eoff = pl.multiple_of(base_e + j * CCHUNK, 8)
            pltpu.make_async_copy(qarr_hbm.at[pl.ds(eoff, CCHUNK)],
                                  pk_v.at[b], isems[b]).wait()
            for u in range(CCHUNK // L):
                p16 = pk_v[b, pl.ds(u * L, L)]
                src_v[b, pl.ds(u * L, L)] = p16 >> 13
                dst_v[b, pl.ds(u * L, L)] = p16 & (2 ** 13 - 1)

        def fire_gather(b):
            pltpu.async_copy(tab_sh.at[src_v.at[b]], rows_v.at[b], gsems[b])

        def wait_gather(b):
            pltpu.make_async_copy(tab_sh.at[src_v.at[b]], rows_v.at[b],
                                  gsems[b]).wait()

        def fire_scatter(b):
            pltpu.async_copy(rows_v.at[b], agg_sh.at[dst_v.at[b]],
                             ssems[b], add=True)

        def wait_scatter(b):
            pltpu.make_async_copy(rows_v.at[b], agg_sh.at[dst_v.at[b]],
                                  ssems[b]).wait()

        def run_region(base_e, n):
            fire_idx(base_e, 0, 0)
            fire_idx(base_e, 1, 1)
            wait_unpack_idx(base_e, 0, 0)
            fire_gather(0)

            def step(j, b, first=False, fire_i=True, fire_g=True):
                if not first:
                    wait_scatter((b + 2) % 3)
                if fire_i:
                    fire_idx(base_e, j + 2, (b + 2) % 3)
                wait_gather(b)
                fire_scatter(b)
                if fire_g:
                    wait_unpack_idx(base_e, j + 1, (b + 1) % 3)
                    fire_gather((b + 1) % 3)

            step(0, 0, first=True)

            def body3(i, carry):
                jb = 1 + 3 * i
                for u in range(3):
                    step(jb + u, (1 + u) % 3)
                return carry

            lax.fori_loop(0, (n - 4) // 3, body3, 0)
            step(n - 3, 1)
            step(n - 2, 2, fire_i=False)
            step(n - 1, 0, fire_i=False, fire_g=False)
            wait_scatter(0)

        for p in range(2):
            pltpu.sync_copy(zeros_hbm.at[pl.ds(row0, TSTRIPE)],
                            agg_sh.at[pl.ds(row0, TSTRIPE)])
            plsc.subcore_barrier()
            q = cid * 2 + p
            for h in range(2):
                c = cnt_v[...][2 * h + p]
                nq = (c + (CCHUNK - 1)) // CCHUNK
                nt = (nq + (NS - 1)) // NS
                n = nt + (1 - nt) % 3
                n = jnp.maximum(n, 4)
                base_e = (h * 4 + q) * CAPQ + sid * (n * CCHUNK)
                run_region(base_e, n)
            plsc.subcore_barrier()
            pltpu.sync_copy(agg_sh.at[pl.ds(row0, TSTRIPE)],
                            out_hbm.at[cid, p, pl.ds(row0, TSTRIPE)])
            plsc.subcore_barrier()

    return k(table, qarr, qcnt, zeros)


def _tc_combine_matmul(partials, wt, b2):
    def body(p_ref, w_ref, b_ref, o_ref):
        x = jnp.concatenate(
            [p_ref[0, 0] + p_ref[1, 0], p_ref[0, 1] + p_ref[1, 1]], axis=0)
        y = jax.lax.dot_general(
            x, w_ref[:], (((1,), (0,)), ((), ())),
            precision=jax.lax.Precision.HIGHEST,
            preferred_element_type=jnp.float32)
        o_ref[:] = jnp.maximum(y + b_ref[:], 0.0)

    return pl.pallas_call(
        body,
        out_shape=jax.ShapeDtypeStruct((AGG_ROWS, DIM), jnp.float32),
    )(partials, wt, b2)


def _sc_batch_gather(ent, rel_emb, norm_emb, head, rel, tail, time):
    mesh = plsc.VectorSubcoreMesh(core_axis_name="c", subcore_axis_name="s")
    out_sds = jax.ShapeDtypeStruct((BATCH, DIM), jnp.float32)

    @functools.partial(
        pl.kernel,
        out_type=(out_sds, out_sds, out_sds, out_sds),
        mesh=mesh,
        scratch_types=[
            pltpu.VMEM((B_PER_W,), jnp.int32),
            pltpu.VMEM((B_PER_W, DIM), jnp.float32),
            pltpu.SemaphoreType.DMA,
        ],
    )
    def k(ent_hbm, rel_hbm, nrm_hbm, hidx_hbm, ridx_hbm, tidx_hbm, midx_hbm,
          h_out, r_out, t_out, n_out, idx_v, rows_v, sem):
        cid = lax.axis_index("c")
        sid = lax.axis_index("s")
        wid = sid * NC + cid
        base = pl.multiple_of(wid * B_PER_W, B_PER_W)

        def one(idx_hbm, table_hbm, out_hbm):
            pltpu.sync_copy(idx_hbm.at[pl.ds(base, B_PER_W)], idx_v)
            pltpu.async_copy(table_hbm.at[idx_v], rows_v, sem).wait()
            pltpu.sync_copy(rows_v, out_hbm.at[pl.ds(base, B_PER_W)])

        one(hidx_hbm, ent_hbm, h_out)
        one(ridx_hbm, rel_hbm, r_out)
        one(tidx_hbm, ent_hbm, t_out)
        one(midx_hbm, nrm_hbm, n_out)

    return k(ent, rel_emb, norm_emb, head, rel, tail, time)


def _tc_score(h, r, t, nv):
    def body(h_ref, r_ref, t_ref, n_ref, o_ref):
        def normalize(x):
            n = jnp.sqrt(jnp.sum(x * x, axis=-1, keepdims=True))
            return x / jnp.maximum(n, 1e-12)

        nvn = normalize(n_ref[:])

        def proj(e):
            return e - jnp.sum(nvn * e, axis=-1, keepdims=True) * nvn

        hh = normalize(proj(h_ref[:]))
        rr = normalize(proj(r_ref[:]))
        tt = normalize(proj(t_ref[:]))
        d = hh + rr - tt
        o_ref[:] = jnp.sqrt(jnp.sum(d * d, axis=-1, keepdims=True))

    return pl.pallas_call(
        body,
        out_shape=jax.ShapeDtypeStruct((BATCH, 1), jnp.float32),
    )(h, r, t, nv)


def kernel(feature, edge_index, head_batched, rel_batched, tail_batched,
           time_batched, W, b, rel_emb, norm_emb):
    n_edges = edge_index.shape[1]
    e_pad = NW * SLICE
    e_half = NS * SLICE
    assert n_edges <= e_pad
    pad = e_pad - n_edges
    src2 = jnp.concatenate([edge_index[0], jnp.zeros((pad,), jnp.int32)])
    dst2 = jnp.concatenate([edge_index[1], jnp.full((pad,), 2 * H, jnp.int32)])

    qa = (src2 >= H).astype(jnp.int32).reshape(NC, e_half)
    qb = (dst2 >= H).astype(jnp.int32).reshape(NC, e_half)
    ind = [(1 - qa) * (1 - qb), (1 - qa) * qb, qa * (1 - qb), qa * qb]
    ranks = [jnp.cumsum(x, axis=1, dtype=jnp.int32) for x in ind]
    qq = qa * 2 + qb
    rsel = jnp.where(qq == 0, ranks[0], jnp.where(qq == 1, ranks[1],
                     jnp.where(qq == 2, ranks[2], ranks[3])))
    pos = (qq * CAPQ + rsel - 1).reshape(-1)
    sloc = src2 - qa.reshape(-1) * H
    dloc = dst2 - qb.reshape(-1) * H
    packed = (sloc << 13) | dloc
    qcnt = jnp.concatenate([
        jnp.concatenate(
            [jnp.stack([ranks[2 * c + p_][h, -1]
                        for h in range(2) for p_ in range(2)]),
             jnp.zeros((L - 4,), jnp.int32)])
        for c in range(NC)])

    zeros = jnp.zeros((H, DIM), jnp.float32)
    wt = W.T
    b2 = b.reshape(1, DIM)
    feature_p = jnp.concatenate(
        [feature, jnp.zeros((AGG_ROWS - ENT, DIM), jnp.float32)])

    trash = jnp.full((4 * CAPQ // NS,), TRASH, jnp.int32)
    qarr = _sc_pos_scatter(packed, pos, trash)
    p1 = _sc_quadrant_aggregate(feature_p, qarr, qcnt, zeros)
    f1 = _tc_combine_matmul(p1, wt, b2)
    p2 = _sc_quadrant_aggregate(f1, qarr, qcnt, zeros)
    ent = _tc_combine_matmul(p2, wt, b2)

    h, r, t, nv = _sc_batch_gather(ent, rel_emb, norm_emb, head_batched,
                                   rel_batched, tail_batched, time_batched)
    return _tc_score(h, r, t, nv).reshape(-1)
